# baseline (speedup 1.0000x reference)
import functools
import jax
import jax.numpy as jnp
from jax import lax
from jax.experimental import pallas as pl
from jax.experimental.pallas import tpu as pltpu

EMBED_DIM = 128
SUB = 32
OFF_Y, OFF_M, OFF_D, OFF_H = 0, 50, 62, 93
TOTAL_ROWS = 117

BLOCK_B = 2048


def _body(ts_ref, tpad_ref, pw_ref, pb_ref, out_ref, comb_ref):
    @pl.when(pl.program_id(0) == 0)
    def _():
        comb_ref[...] = lax.dot_general(
            tpad_ref[...],
            pw_ref[...],
            (((1,), (1,)), ((), ())),
            preferred_element_type=jnp.float32,
        )

    idx = ts_ref[...]
    cols = lax.broadcasted_iota(jnp.int32, (idx.shape[0], EMBED_DIM), 1)
    hot = (
        (cols == idx[:, 0:1] + OFF_Y)
        | (cols == idx[:, 1:2] + OFF_M)
        | (cols == idx[:, 2:3] + OFF_D)
        | (cols == idx[:, 3:4] + OFF_H)
    ).astype(jnp.float32)
    out_ref[...] = (
        jnp.dot(hot, comb_ref[...], preferred_element_type=jnp.float32) + pb_ref[...]
    )


def kernel(timestamps, year_table, month_table, day_table, hour_table, proj_w, proj_b):
    B = timestamps.shape[0]
    ts = timestamps.astype(jnp.int32)

    tpad = jnp.zeros((EMBED_DIM, EMBED_DIM), dtype=jnp.float32)
    tpad = tpad.at[OFF_Y : OFF_Y + 50, 0 * SUB : 1 * SUB].set(year_table)
    tpad = tpad.at[OFF_M : OFF_M + 12, 1 * SUB : 2 * SUB].set(month_table)
    tpad = tpad.at[OFF_D : OFF_D + 31, 2 * SUB : 3 * SUB].set(day_table)
    tpad = tpad.at[OFF_H : OFF_H + 24, 3 * SUB : 4 * SUB].set(hour_table)

    grid = (B // BLOCK_B,)
    return pl.pallas_call(
        _body,
        grid=grid,
        in_specs=[
            pl.BlockSpec((BLOCK_B, 4), lambda i: (i, 0)),
            pl.BlockSpec((EMBED_DIM, EMBED_DIM), lambda i: (0, 0)),
            pl.BlockSpec((EMBED_DIM, EMBED_DIM), lambda i: (0, 0)),
            pl.BlockSpec((1, EMBED_DIM), lambda i: (0, 0)),
        ],
        out_specs=pl.BlockSpec((BLOCK_B, EMBED_DIM), lambda i: (i, 0)),
        out_shape=jax.ShapeDtypeStruct((B, EMBED_DIM), jnp.float32),
        scratch_shapes=[pltpu.VMEM((EMBED_DIM, EMBED_DIM), jnp.float32)],
    )(ts, tpad, proj_w, proj_b.reshape(1, EMBED_DIM))


# speedup vs baseline: 1.2801x; 1.2801x over previous
import jax
import jax.numpy as jnp
from jax import lax
from jax.experimental import pallas as pl
from jax.experimental.pallas import tpu as pltpu

EMBED_DIM = 128
SUB = 32
N_Y, N_M, N_D, N_H = 50, 12, 31, 24

BLOCK_B = 2048


def _band_dot(table, pw, f):
    return lax.dot_general(
        table,
        pw[:, f * SUB : (f + 1) * SUB],
        (((1,), (1,)), ((), ())),
        preferred_element_type=jnp.float32,
    )


def _body(ts_ref, y_ref, m_ref, d_ref, h_ref, pw_ref, pb_ref, out_ref,
          cy_ref, cm_ref, cd_ref, ch_ref):
    @pl.when(pl.program_id(0) == 0)
    def _():
        pw = pw_ref[...]
        cy_ref[...] = _band_dot(y_ref[...], pw, 0)
        cm_ref[...] = _band_dot(m_ref[...], pw, 1)
        cd_ref[...] = _band_dot(d_ref[...], pw, 2)
        ch_ref[...] = _band_dot(h_ref[...], pw, 3) + pb_ref[...]

    idx = ts_ref[...]
    nb = idx.shape[0]

    def hot(col, n):
        cols = lax.broadcasted_iota(jnp.int32, (nb, n), 1)
        return (cols == idx[:, col : col + 1]).astype(jnp.float32)

    acc = jnp.dot(hot(0, N_Y), cy_ref[...], preferred_element_type=jnp.float32)
    acc += jnp.dot(hot(1, N_M), cm_ref[...], preferred_element_type=jnp.float32)
    acc += jnp.dot(hot(2, N_D), cd_ref[...], preferred_element_type=jnp.float32)
    acc += jnp.dot(hot(3, N_H), ch_ref[...], preferred_element_type=jnp.float32)
    out_ref[...] = acc


def kernel(timestamps, year_table, month_table, day_table, hour_table, proj_w, proj_b):
    B = timestamps.shape[0]
    if timestamps.dtype != jnp.int32:
        timestamps = timestamps.astype(jnp.int32)

    grid = (B // BLOCK_B,)
    full = lambda r, c: pl.BlockSpec((r, c), lambda i: (0, 0))
    return pl.pallas_call(
        _body,
        grid=grid,
        in_specs=[
            pl.BlockSpec((BLOCK_B, 4), lambda i: (i, 0)),
            full(N_Y, SUB),
            full(N_M, SUB),
            full(N_D, SUB),
            full(N_H, SUB),
            full(EMBED_DIM, EMBED_DIM),
            full(1, EMBED_DIM),
        ],
        out_specs=pl.BlockSpec((BLOCK_B, EMBED_DIM), lambda i: (i, 0)),
        out_shape=jax.ShapeDtypeStruct((B, EMBED_DIM), jnp.float32),
        scratch_shapes=[
            pltpu.VMEM((N_Y, EMBED_DIM), jnp.float32),
            pltpu.VMEM((N_M, EMBED_DIM), jnp.float32),
            pltpu.VMEM((N_D, EMBED_DIM), jnp.float32),
            pltpu.VMEM((N_H, EMBED_DIM), jnp.float32),
        ],
    )(
        timestamps,
        year_table,
        month_table,
        day_table,
        hour_table,
        proj_w,
        proj_b.reshape(1, EMBED_DIM),
    )
